# Initial kernel scaffold; baseline (speedup 1.0000x reference)
import jax
import jax.numpy as jnp
from jax.experimental import pallas as pl


def kernel(patches):
    raise NotImplementedError("write your pallas kernel here")


# speedup vs baseline: 139.2903x; 139.2903x over previous
import functools

import jax
import jax.numpy as jnp
from jax import lax
from jax.experimental import pallas as pl
from jax.experimental.pallas import tpu as pltpu
from jax.experimental.pallas import tpu_sc as plsc

_NC = 2
_NS = 16
_NW = _NC * _NS


def kernel(patches):
    batch = patches.shape[0]
    G = batch * 8
    gpw = G // _NW

    in4 = patches.reshape(G, 8, 64, 192)

    mesh = plsc.VectorSubcoreMesh(core_axis_name="c", subcore_axis_name="s")

    @functools.partial(
        pl.kernel,
        mesh=mesh,
        out_type=jax.ShapeDtypeStruct((G * 64, 8 * 192), jnp.float32),
        scratch_types=[pltpu.VMEM((8, 64, 192), jnp.float32)],
        compiler_params=pltpu.CompilerParams(use_tc_tiling_on_sc=False),
    )
    def unpatch(in_hbm, out_hbm, buf):
        wid = lax.axis_index("s") * _NC + lax.axis_index("c")
        g0 = wid * gpw

        def body(t, carry):
            g = g0 + t
            pltpu.sync_copy(in_hbm.at[g], buf)
            row0 = g * 64
            for i in range(8):
                pltpu.sync_copy(
                    buf.at[i],
                    out_hbm.at[pl.ds(row0, 64), pl.ds(i * 192, 192)],
                )
            return carry

        lax.fori_loop(0, gpw, body, 0)

    out = unpatch(in4)
    return out.reshape(batch, 512, 512, 3)
